# Initial kernel scaffold; baseline (speedup 1.0000x reference)
import jax
import jax.numpy as jnp
from jax.experimental import pallas as pl


def kernel(edge_index, edge_attr, num_nodes):
    raise NotImplementedError("write your pallas kernel here")


# speedup vs baseline: 1.2437x; 1.2437x over previous
import jax
import jax.numpy as jnp
from jax.experimental import pallas as pl


def _rewire_kernel(ei_ref, ea_ref, ei_out, ea_out):
    ei_out[...] = ei_ref[...]
    ea_out[...] = ea_ref[...]


def kernel(edge_index, edge_attr, num_nodes):
    del num_nodes
    out = pl.pallas_call(
        _rewire_kernel,
        out_shape=(
            jax.ShapeDtypeStruct(edge_index.shape, edge_index.dtype),
            jax.ShapeDtypeStruct(edge_attr.shape, edge_attr.dtype),
        ),
    )(edge_index, edge_attr)
    return out
